# baseline (speedup 1.0000x reference)
import functools

import jax
import jax.numpy as jnp
from jax import lax
from jax.experimental import pallas as pl
from jax.experimental.pallas import tpu as pltpu
from jax.experimental.pallas import tpu_sc as plsc

N_NODES = 10000
NODE_DIM = 128
EDGE_DIM = 16
HIDDEN_DIM = 128
N_EDGES = 320000

NC = 2
NS = 16
NW = NC * NS
P = 5
EP = N_EDGES // P
EPW = EP // NW
CH = 80
STRIPE = 1000

KG = 5
OUTER_G = EPW // (KG * CH)

KS = 4
OUTER_S = EPW // (KS * CH)
TAIL_S = EPW - OUTER_S * KS * CH


@functools.cache
def _make_sc_gather():
    mesh = plsc.VectorSubcoreMesh(core_axis_name="c", subcore_axis_name="s")
    return pl.kernel(
        _sc_gather_body,
        mesh=mesh,
        out_type=jax.ShapeDtypeStruct((EP, NODE_DIM), jnp.float32),
        scratch_types=[
            pltpu.VMEM((KG * CH,), jnp.int32),
            pltpu.VMEM((KG * CH, NODE_DIM), jnp.float32),
            pltpu.SemaphoreType.DMA,
        ],
    )


def _sc_gather_body(x_hbm, src_hbm, out_hbm, idx_v, rows_v, sem):
    wid = lax.axis_index("s") * NC + lax.axis_index("c")

    def step(base, k):
        pltpu.sync_copy(src_hbm.at[pl.ds(base, k * CH)], idx_v.at[pl.ds(0, k * CH)])
        cps = [
            pltpu.async_copy(
                x_hbm.at[idx_v.at[pl.ds(t * CH, CH)]],
                rows_v.at[pl.ds(t * CH, CH)],
                sem,
            )
            for t in range(k)
        ]
        for cp in cps:
            cp.wait()
        pltpu.sync_copy(rows_v.at[pl.ds(0, k * CH)], out_hbm.at[pl.ds(base, k * CH)])

    def body(j, carry):
        step(wid * EPW + j * (KG * CH), KG)
        return carry

    lax.fori_loop(0, OUTER_G, body, 0)


@functools.cache
def _make_sc_scatter():
    mesh = plsc.VectorSubcoreMesh(core_axis_name="c", subcore_axis_name="s")
    return pl.kernel(
        _sc_scatter_body,
        mesh=mesh,
        out_type=jax.ShapeDtypeStruct((NC * N_NODES, HIDDEN_DIM), jnp.float32),
        scratch_types=[
            [pltpu.VMEM((CH,), jnp.int32) for _ in range(KS)],
            pltpu.VMEM((KS * CH, HIDDEN_DIM), jnp.float32),
            pltpu.VMEM_SHARED((N_NODES, HIDDEN_DIM), jnp.float32),
            pltpu.SemaphoreType.DMA,
        ],
    )


def _sc_scatter_body(
    msg_hbm, dst_hbm, zero_hbm, out_hbm, idx_vs, rows_v, agg_sh, isem
):
    cid = lax.axis_index("c")
    sid = lax.axis_index("s")
    wid = sid * NC + cid
    @pl.when(sid < N_NODES // STRIPE)
    def _():
        pltpu.sync_copy(
            zero_hbm.at[pl.ds(sid * STRIPE, STRIPE)],
            agg_sh.at[pl.ds(sid * STRIPE, STRIPE)],
        )

    plsc.subcore_barrier()

    def step(base, k):
        icps = [
            pltpu.async_copy(
                dst_hbm.at[pl.ds(base + t * CH, CH)], idx_vs[t], isem
            )
            for t in range(k)
        ]
        pltpu.sync_copy(msg_hbm.at[pl.ds(base, k * CH)], rows_v.at[pl.ds(0, k * CH)])
        for cp in icps:
            cp.wait()
        for t in range(k):
            pltpu.sync_copy(
                rows_v.at[pl.ds(t * CH, CH)], agg_sh.at[idx_vs[t]], add=True
            )

    def body(j, carry):
        step(wid * EPW + j * (KS * CH), KS)
        return carry

    lax.fori_loop(0, OUTER_S, body, 0)
    step(wid * EPW + OUTER_S * (KS * CH), TAIL_S // CH)
    plsc.subcore_barrier()

    @pl.when(sid < N_NODES // STRIPE)
    def _():
        pltpu.sync_copy(
            agg_sh.at[pl.ds(sid * STRIPE, STRIPE)],
            out_hbm.at[pl.ds(cid * N_NODES + sid * STRIPE, STRIPE)],
        )


def _mlp_body(gx_ref, ea_ref, w1x_ref, w1e_ref, b1_ref, w2_ref, b2_ref, out_ref):
    gx = gx_ref[...].astype(jnp.bfloat16)
    ea_t = ea_ref[...].astype(jnp.bfloat16)
    h = jnp.dot(gx, w1x_ref[...], preferred_element_type=jnp.float32)
    h = h + lax.dot_general(
        ea_t,
        w1e_ref[...],
        dimension_numbers=(((0,), (0,)), ((), ())),
        preferred_element_type=jnp.float32,
    )
    h = jnp.maximum(h + b1_ref[...], 0.0).astype(jnp.bfloat16)
    out_ref[...] = (
        jnp.dot(h, w2_ref[...], preferred_element_type=jnp.float32) + b2_ref[...]
    )


def _gru_body(*refs):
    part_refs = refs[: 2 * P]
    x_ref, wih_ref, whh_ref, bih_ref, bhh_ref, out_ref = refs[2 * P :]
    agg = part_refs[0][...]
    for r in part_refs[1:]:
        agg = agg + r[...]
    x = x_ref[...]
    gi = jnp.dot(agg, wih_ref[...], preferred_element_type=jnp.float32) + bih_ref[...]
    gh = jnp.dot(x, whh_ref[...], preferred_element_type=jnp.float32) + bhh_ref[...]
    i_r = gi[:, :NODE_DIM]
    i_z = gi[:, NODE_DIM : 2 * NODE_DIM]
    i_n = gi[:, 2 * NODE_DIM :]
    h_r = gh[:, :NODE_DIM]
    h_z = gh[:, NODE_DIM : 2 * NODE_DIM]
    h_n = gh[:, 2 * NODE_DIM :]
    r = jax.nn.sigmoid(i_r + h_r)
    z = jax.nn.sigmoid(i_z + h_z)
    n = jnp.tanh(i_n + r * h_n)
    out_ref[...] = (1.0 - z) * n + z * x


BE = 6400
BN = 2000


def kernel(x, edge_index, edge_attr, W1, b1, W2, b2, W_ih, b_ih, W_hh, b_hh):
    src = edge_index[0].astype(jnp.int32)
    dst = edge_index[1].astype(jnp.int32)
    ea_t = edge_attr.T

    w1x_t = W1[:, :NODE_DIM].T.astype(jnp.bfloat16)
    w1e_t = W1[:, NODE_DIM:].T.astype(jnp.bfloat16)
    w2_t = W2.T.astype(jnp.bfloat16)
    zero = jnp.zeros((N_NODES, HIDDEN_DIM), jnp.float32)

    gather = _make_sc_gather()
    scatter = _make_sc_scatter()

    def mlp(gathered, phase):
        return pl.pallas_call(
            _mlp_body,
            grid=(EP // BE,),
            in_specs=[
                pl.BlockSpec((BE, NODE_DIM), lambda i: (i, 0)),
                pl.BlockSpec(
                    (EDGE_DIM, BE), lambda i, p=phase: (0, p * (EP // BE) + i)
                ),
                pl.BlockSpec((NODE_DIM, HIDDEN_DIM), lambda i: (0, 0)),
                pl.BlockSpec((EDGE_DIM, HIDDEN_DIM), lambda i: (0, 0)),
                pl.BlockSpec((1, HIDDEN_DIM), lambda i: (0, 0)),
                pl.BlockSpec((HIDDEN_DIM, HIDDEN_DIM), lambda i: (0, 0)),
                pl.BlockSpec((1, HIDDEN_DIM), lambda i: (0, 0)),
            ],
            out_specs=pl.BlockSpec((BE, HIDDEN_DIM), lambda i: (i, 0)),
            out_shape=jax.ShapeDtypeStruct((EP, HIDDEN_DIM), jnp.float32),
        )(
            gathered,
            ea_t,
            w1x_t,
            w1e_t,
            b1.reshape(1, HIDDEN_DIM),
            w2_t,
            b2.reshape(1, HIDDEN_DIM),
        )

    partials = []
    for p in range(P):
        src_p = lax.dynamic_slice_in_dim(src, p * EP, EP)
        dst_p = lax.dynamic_slice_in_dim(dst, p * EP, EP)
        gathered = gather(x, src_p)
        messages = mlp(gathered, p)
        agg2 = scatter(messages, dst_p, zero)
        partials.append(agg2)

    gru_in_specs = []
    gru_args = []
    for agg2 in partials:
        for half in range(NC):
            gru_in_specs.append(
                pl.BlockSpec(
                    (BN, HIDDEN_DIM),
                    lambda i, h=half: (h * (N_NODES // BN) + i, 0),
                )
            )
            gru_args.append(agg2)
    gru_in_specs += [
        pl.BlockSpec((BN, NODE_DIM), lambda i: (i, 0)),
        pl.BlockSpec((HIDDEN_DIM, 3 * NODE_DIM), lambda i: (0, 0)),
        pl.BlockSpec((NODE_DIM, 3 * NODE_DIM), lambda i: (0, 0)),
        pl.BlockSpec((1, 3 * NODE_DIM), lambda i: (0, 0)),
        pl.BlockSpec((1, 3 * NODE_DIM), lambda i: (0, 0)),
    ]
    gru_args += [
        x,
        W_ih.T,
        W_hh.T,
        b_ih.reshape(1, 3 * NODE_DIM),
        b_hh.reshape(1, 3 * NODE_DIM),
    ]

    x_new = pl.pallas_call(
        _gru_body,
        grid=(N_NODES // BN,),
        in_specs=gru_in_specs,
        out_specs=pl.BlockSpec((BN, NODE_DIM), lambda i: (i, 0)),
        out_shape=jax.ShapeDtypeStruct((N_NODES, NODE_DIM), jnp.float32),
    )(*gru_args)
    return x_new


# speedup vs baseline: 1.2012x; 1.2012x over previous
import functools

import jax
import jax.numpy as jnp
from jax import lax
from jax.experimental import pallas as pl
from jax.experimental.pallas import tpu as pltpu
from jax.experimental.pallas import tpu_sc as plsc

N_NODES = 10000
NODE_DIM = 128
EDGE_DIM = 16
HIDDEN_DIM = 128
N_EDGES = 320000

NC = 2
NS = 16
NW = NC * NS
P = 2
EP = N_EDGES // P
EPW = EP // NW
CH = 80
STRIPE = 1000

KG = 6
OUTER_G = 10
TAIL = (CH, CH, CH - 40)

KS = 4
OUTER_S = 15


@functools.cache
def _make_sc_gather():
    mesh = plsc.VectorSubcoreMesh(core_axis_name="c", subcore_axis_name="s")
    return pl.kernel(
        _sc_gather_body,
        mesh=mesh,
        out_type=jax.ShapeDtypeStruct((EP, NODE_DIM), jnp.float32),
        scratch_types=[
            pltpu.VMEM((KG * CH,), jnp.int32),
            pltpu.VMEM((KG * CH, NODE_DIM), jnp.float32),
            pltpu.SemaphoreType.DMA,
        ],
    )


def _sc_gather_body(x_hbm, src_hbm, out_hbm, idx_v, rows_v, sem):
    wid = lax.axis_index("s") * NC + lax.axis_index("c")

    def step(base, chunks):
        n = sum(chunks)
        pltpu.sync_copy(src_hbm.at[pl.ds(base, n)], idx_v.at[pl.ds(0, n)])
        cps, off = [], 0
        for c in chunks:
            cps.append(
                pltpu.async_copy(
                    x_hbm.at[idx_v.at[pl.ds(off, c)]],
                    rows_v.at[pl.ds(off, c)],
                    sem,
                )
            )
            off += c
        for cp in cps:
            cp.wait()
        pltpu.sync_copy(rows_v.at[pl.ds(0, n)], out_hbm.at[pl.ds(base, n)])

    def body(j, carry):
        step(wid * EPW + j * (KG * CH), (CH,) * KG)
        return carry

    lax.fori_loop(0, OUTER_G, body, 0)
    step(wid * EPW + OUTER_G * KG * CH, TAIL)


@functools.cache
def _make_sc_scatter():
    mesh = plsc.VectorSubcoreMesh(core_axis_name="c", subcore_axis_name="s")
    return pl.kernel(
        _sc_scatter_body,
        mesh=mesh,
        out_type=jax.ShapeDtypeStruct((NC * N_NODES, HIDDEN_DIM), jnp.float32),
        scratch_types=[
            [pltpu.VMEM((CH,), jnp.int32) for _ in range(KS)],
            pltpu.VMEM((CH - 40,), jnp.int32),
            pltpu.VMEM((KS * CH, HIDDEN_DIM), jnp.float32),
            pltpu.VMEM_SHARED((N_NODES, HIDDEN_DIM), jnp.float32),
            pltpu.SemaphoreType.DMA,
        ],
    )


def _sc_scatter_body(
    msg_hbm, dst_hbm, zero_hbm, out_hbm, idx_vs, idx_t, rows_v, agg_sh, isem
):
    cid = lax.axis_index("c")
    sid = lax.axis_index("s")
    wid = sid * NC + cid
    @pl.when(sid < N_NODES // STRIPE)
    def _():
        pltpu.sync_copy(
            zero_hbm.at[pl.ds(sid * STRIPE, STRIPE)],
            agg_sh.at[pl.ds(sid * STRIPE, STRIPE)],
        )

    plsc.subcore_barrier()

    def step(base, bufs):
        n = sum(b.shape[0] for b in bufs)
        icps, off = [], 0
        for b in bufs:
            icps.append(
                pltpu.async_copy(dst_hbm.at[pl.ds(base + off, b.shape[0])], b, isem)
            )
            off += b.shape[0]
        pltpu.sync_copy(msg_hbm.at[pl.ds(base, n)], rows_v.at[pl.ds(0, n)])
        for cp in icps:
            cp.wait()
        off = 0
        for b in bufs:
            pltpu.sync_copy(
                rows_v.at[pl.ds(off, b.shape[0])], agg_sh.at[b], add=True
            )
            off += b.shape[0]

    def body(j, carry):
        step(wid * EPW + j * (KS * CH), idx_vs)
        return carry

    lax.fori_loop(0, OUTER_S, body, 0)
    step(wid * EPW + OUTER_S * (KS * CH), [idx_vs[0], idx_vs[1], idx_t])
    plsc.subcore_barrier()

    @pl.when(sid < N_NODES // STRIPE)
    def _():
        pltpu.sync_copy(
            agg_sh.at[pl.ds(sid * STRIPE, STRIPE)],
            out_hbm.at[pl.ds(cid * N_NODES + sid * STRIPE, STRIPE)],
        )


def _mlp_body(gx_ref, ea_ref, w1x_ref, w1e_ref, b1_ref, w2_ref, b2_ref, out_ref):
    gx = gx_ref[...].astype(jnp.bfloat16)
    ea_t = ea_ref[...].astype(jnp.bfloat16)
    h = jnp.dot(gx, w1x_ref[...], preferred_element_type=jnp.float32)
    h = h + lax.dot_general(
        ea_t,
        w1e_ref[...],
        dimension_numbers=(((0,), (0,)), ((), ())),
        preferred_element_type=jnp.float32,
    )
    h = jnp.maximum(h + b1_ref[...], 0.0).astype(jnp.bfloat16)
    out_ref[...] = (
        jnp.dot(h, w2_ref[...], preferred_element_type=jnp.float32) + b2_ref[...]
    )


def _gru_body(*refs):
    part_refs = refs[: 2 * P]
    x_ref, wih_ref, whh_ref, bih_ref, bhh_ref, out_ref = refs[2 * P :]
    agg = part_refs[0][...]
    for r in part_refs[1:]:
        agg = agg + r[...]
    x = x_ref[...]
    gi = jnp.dot(agg, wih_ref[...], preferred_element_type=jnp.float32) + bih_ref[...]
    gh = jnp.dot(x, whh_ref[...], preferred_element_type=jnp.float32) + bhh_ref[...]
    i_r = gi[:, :NODE_DIM]
    i_z = gi[:, NODE_DIM : 2 * NODE_DIM]
    i_n = gi[:, 2 * NODE_DIM :]
    h_r = gh[:, :NODE_DIM]
    h_z = gh[:, NODE_DIM : 2 * NODE_DIM]
    h_n = gh[:, 2 * NODE_DIM :]
    r = jax.nn.sigmoid(i_r + h_r)
    z = jax.nn.sigmoid(i_z + h_z)
    n = jnp.tanh(i_n + r * h_n)
    out_ref[...] = (1.0 - z) * n + z * x


BE = 6400
BN = 2000


def kernel(x, edge_index, edge_attr, W1, b1, W2, b2, W_ih, b_ih, W_hh, b_hh):
    src = edge_index[0].astype(jnp.int32)
    dst = edge_index[1].astype(jnp.int32)
    ea_t = edge_attr.T

    w1x_t = W1[:, :NODE_DIM].T.astype(jnp.bfloat16)
    w1e_t = W1[:, NODE_DIM:].T.astype(jnp.bfloat16)
    w2_t = W2.T.astype(jnp.bfloat16)
    zero = jnp.zeros((N_NODES, HIDDEN_DIM), jnp.float32)

    gather = _make_sc_gather()
    scatter = _make_sc_scatter()

    def mlp(gathered, phase):
        return pl.pallas_call(
            _mlp_body,
            grid=(EP // BE,),
            in_specs=[
                pl.BlockSpec((BE, NODE_DIM), lambda i: (i, 0)),
                pl.BlockSpec(
                    (EDGE_DIM, BE), lambda i, p=phase: (0, p * (EP // BE) + i)
                ),
                pl.BlockSpec((NODE_DIM, HIDDEN_DIM), lambda i: (0, 0)),
                pl.BlockSpec((EDGE_DIM, HIDDEN_DIM), lambda i: (0, 0)),
                pl.BlockSpec((1, HIDDEN_DIM), lambda i: (0, 0)),
                pl.BlockSpec((HIDDEN_DIM, HIDDEN_DIM), lambda i: (0, 0)),
                pl.BlockSpec((1, HIDDEN_DIM), lambda i: (0, 0)),
            ],
            out_specs=pl.BlockSpec((BE, HIDDEN_DIM), lambda i: (i, 0)),
            out_shape=jax.ShapeDtypeStruct((EP, HIDDEN_DIM), jnp.float32),
        )(
            gathered,
            ea_t,
            w1x_t,
            w1e_t,
            b1.reshape(1, HIDDEN_DIM),
            w2_t,
            b2.reshape(1, HIDDEN_DIM),
        )

    partials = []
    for p in range(P):
        src_p = lax.dynamic_slice_in_dim(src, p * EP, EP)
        dst_p = lax.dynamic_slice_in_dim(dst, p * EP, EP)
        gathered = gather(x, src_p)
        messages = mlp(gathered, p)
        agg2 = scatter(messages, dst_p, zero)
        partials.append(agg2)

    gru_in_specs = []
    gru_args = []
    for agg2 in partials:
        for half in range(NC):
            gru_in_specs.append(
                pl.BlockSpec(
                    (BN, HIDDEN_DIM),
                    lambda i, h=half: (h * (N_NODES // BN) + i, 0),
                )
            )
            gru_args.append(agg2)
    gru_in_specs += [
        pl.BlockSpec((BN, NODE_DIM), lambda i: (i, 0)),
        pl.BlockSpec((HIDDEN_DIM, 3 * NODE_DIM), lambda i: (0, 0)),
        pl.BlockSpec((NODE_DIM, 3 * NODE_DIM), lambda i: (0, 0)),
        pl.BlockSpec((1, 3 * NODE_DIM), lambda i: (0, 0)),
        pl.BlockSpec((1, 3 * NODE_DIM), lambda i: (0, 0)),
    ]
    gru_args += [
        x,
        W_ih.T,
        W_hh.T,
        b_ih.reshape(1, 3 * NODE_DIM),
        b_hh.reshape(1, 3 * NODE_DIM),
    ]

    x_new = pl.pallas_call(
        _gru_body,
        grid=(N_NODES // BN,),
        in_specs=gru_in_specs,
        out_specs=pl.BlockSpec((BN, NODE_DIM), lambda i: (i, 0)),
        out_shape=jax.ShapeDtypeStruct((N_NODES, NODE_DIM), jnp.float32),
    )(*gru_args)
    return x_new
